# Initial kernel scaffold; baseline (speedup 1.0000x reference)
import jax
import jax.numpy as jnp
from jax.experimental import pallas as pl


def kernel(var_node_features, con_node_features, edge_index_var, edge_index_con, edge_features_var, edge_features_con, rhs, index, obj, params):
    raise NotImplementedError("write your pallas kernel here")


# speedup vs baseline: 1.7002x; 1.7002x over previous
import functools

import jax
import jax.numpy as jnp
from jax import lax
from jax.experimental import pallas as pl
from jax.experimental.pallas import tpu as pltpu
from jax.experimental.pallas import tpu_sc as plsc

H = 128
BN_EPS = 1e-5
F32 = jnp.float32


def _row2(b):
    return b.reshape(1, -1)


def _mlp2(p, x, br=1000):
    n, fi = x.shape

    def body(x_ref, w1, b1, w2, b2, o_ref):
        h = jnp.maximum(jnp.dot(x_ref[...], w1[...],
                                preferred_element_type=F32) + b1[...], 0.0)
        o_ref[...] = jnp.dot(h, w2[...], preferred_element_type=F32) + b2[...]

    return pl.pallas_call(
        body,
        grid=(n // br,),
        in_specs=[
            pl.BlockSpec((br, fi), lambda i: (i, 0)),
            pl.BlockSpec((fi, H), lambda i: (0, 0)),
            pl.BlockSpec((1, H), lambda i: (0, 0)),
            pl.BlockSpec((H, H), lambda i: (0, 0)),
            pl.BlockSpec((1, H), lambda i: (0, 0)),
        ],
        out_specs=pl.BlockSpec((br, H), lambda i: (i, 0)),
        out_shape=jax.ShapeDtypeStruct((n, H), F32),
    )(x, p["l1"]["w"], _row2(p["l1"]["b"]), p["l2"]["w"], _row2(p["l2"]["b"]))


def _assign(p, x, br=1000):
    n = x.shape[0]

    def body(x_ref, w1, b1, w2, b2, o_ref):
        h = jnp.maximum(jnp.dot(x_ref[...], w1[...],
                                preferred_element_type=F32) + b1[...], 0.0)
        z = jnp.dot(h, w2[...], preferred_element_type=F32) + b2[...]
        o_ref[...] = jax.nn.sigmoid(z)

    return pl.pallas_call(
        body,
        grid=(n // br,),
        in_specs=[
            pl.BlockSpec((br, H), lambda i: (i, 0)),
            pl.BlockSpec((H, H), lambda i: (0, 0)),
            pl.BlockSpec((1, H), lambda i: (0, 0)),
            pl.BlockSpec((H, 1), lambda i: (0, 0)),
            pl.BlockSpec((1, 1), lambda i: (0, 0)),
        ],
        out_specs=pl.BlockSpec((br, 1), lambda i: (i, 0)),
        out_shape=jax.ShapeDtypeStruct((n, 1), F32),
    )(x, p["l1"]["w"], _row2(p["l1"]["b"]), p["l2"]["w"], _row2(p["l2"]["b"]))


def _stats_body(h2, i, h2_ref, st_ref):
    h2_ref[...] = h2
    s = jnp.concatenate([jnp.sum(h2, 0, keepdims=True),
                         jnp.sum(h2 * h2, 0, keepdims=True)], axis=0)

    @pl.when(i == 0)
    def _():
        st_ref[...] = jnp.zeros_like(st_ref)

    st_ref[...] += s


def _seqbn_joint_stats(xa, xb, w1, b1, w2, b2, br=1000):
    n = xa.shape[0]

    def body(xa_ref, xb_ref, w1_ref, b1_ref, w2_ref, b2_ref,
             h2_ref, st_ref):
        i = pl.program_id(0)
        x = jnp.concatenate([xa_ref[...], xb_ref[...]], axis=1)
        h1 = jnp.maximum(jnp.dot(x, w1_ref[...], preferred_element_type=F32)
                         + b1_ref[...], 0.0)
        h2 = jnp.maximum(jnp.dot(h1, w2_ref[...], preferred_element_type=F32)
                         + b2_ref[...], 0.0)
        _stats_body(h2, i, h2_ref, st_ref)

    return pl.pallas_call(
        body,
        grid=(n // br,),
        in_specs=[
            pl.BlockSpec((br, H), lambda i: (i, 0)),
            pl.BlockSpec((br, 1), lambda i: (i, 0)),
            pl.BlockSpec((H + 1, H), lambda i: (0, 0)),
            pl.BlockSpec((1, H), lambda i: (0, 0)),
            pl.BlockSpec((H, H), lambda i: (0, 0)),
            pl.BlockSpec((1, H), lambda i: (0, 0)),
        ],
        out_specs=[
            pl.BlockSpec((br, H), lambda i: (i, 0)),
            pl.BlockSpec((2, H), lambda i: (0, 0)),
        ],
        out_shape=[jax.ShapeDtypeStruct((n, H), F32),
                   jax.ShapeDtypeStruct((2, H), F32)],
    )(xa, xb, w1, b1, w2, b2)


def _seqbn_mlp_stats(t, tmpa, tmpb, scal, w1, b1, w2, b2, br=1000):
    n = t.shape[0]

    def body(t_ref, ta_ref, tb_ref, s_ref, w1_ref, b1_ref, w2_ref, b2_ref,
             h2_ref, st_ref):
        i = pl.program_id(0)
        tmp = jnp.concatenate([ta_ref[...], tb_ref[...]], axis=1)
        x = s_ref[0, 0] * t_ref[...] + tmp
        h1 = jnp.maximum(jnp.dot(x, w1_ref[...], preferred_element_type=F32)
                         + b1_ref[...], 0.0)
        h2 = jnp.maximum(jnp.dot(h1, w2_ref[...], preferred_element_type=F32)
                         + b2_ref[...], 0.0)
        _stats_body(h2, i, h2_ref, st_ref)

    return pl.pallas_call(
        body,
        grid=(n // br,),
        in_specs=[
            pl.BlockSpec((br, H), lambda i: (i, 0)),
            pl.BlockSpec((br, H // 2), lambda i: (i, 0)),
            pl.BlockSpec((br, H // 2), lambda i: (i, 0)),
            pl.BlockSpec((1, 1), lambda i: (0, 0)),
            pl.BlockSpec((H, H), lambda i: (0, 0)),
            pl.BlockSpec((1, H), lambda i: (0, 0)),
            pl.BlockSpec((H, H), lambda i: (0, 0)),
            pl.BlockSpec((1, H), lambda i: (0, 0)),
        ],
        out_specs=[
            pl.BlockSpec((br, H), lambda i: (i, 0)),
            pl.BlockSpec((2, H), lambda i: (0, 0)),
        ],
        out_shape=[jax.ShapeDtypeStruct((n, H), F32),
                   jax.ShapeDtypeStruct((2, H), F32)],
    )(t, tmpa, tmpb, scal, w1, b1, w2, b2)


def _bn_norm(h2, mu, var, g, b, relu_out=False, ecol=None,
             split=False, br=1000):
    n = h2.shape[0]

    def body(*refs):
        refs = list(refs)
        h2_ref, mu_ref, var_ref, g_ref, b_ref = refs[:5]
        pos = 5
        e_ref = None
        if ecol is not None:
            e_ref = refs[pos]
            pos += 1
        orefs = refs[pos:]
        y = (h2_ref[...] - mu_ref[...]) / jnp.sqrt(var_ref[...] + BN_EPS)
        y = y * g_ref[...] + b_ref[...]
        if relu_out:
            y = jnp.maximum(y, 0.0)
        if ecol is not None:
            col = lax.broadcasted_iota(jnp.int32, (br, H), 1)
            y = jnp.where(col == H - 1, e_ref[...], y)
        if split:
            orefs[0][...] = y[:, : H // 2]
            orefs[1][...] = y[:, H // 2:]
        else:
            orefs[0][...] = y

    in_specs = [
        pl.BlockSpec((br, H), lambda i: (i, 0)),
        pl.BlockSpec((1, H), lambda i: (0, 0)),
        pl.BlockSpec((1, H), lambda i: (0, 0)),
        pl.BlockSpec((1, H), lambda i: (0, 0)),
        pl.BlockSpec((1, H), lambda i: (0, 0)),
    ]
    args = [h2, mu, var, g, b]
    if ecol is not None:
        in_specs.append(pl.BlockSpec((br, 1), lambda i: (i, 0)))
        args.append(ecol)
    if split:
        out_specs = [pl.BlockSpec((br, H // 2), lambda i: (i, 0))] * 2
        out_shape = [jax.ShapeDtypeStruct((n, H // 2), F32)] * 2
    else:
        out_specs = pl.BlockSpec((br, H), lambda i: (i, 0))
        out_shape = jax.ShapeDtypeStruct((n, H), F32)
    return pl.pallas_call(
        body,
        grid=(n // br,),
        in_specs=in_specs,
        out_specs=out_specs,
        out_shape=out_shape,
    )(*args)


def _edge_encode(p, ea, be=2000):
    ne = ea.shape[0]
    w1 = p["l1"]["w"]
    b1 = _row2(p["l1"]["b"])
    w2 = p["l2"]["w"]
    b2 = _row2(p["l2"]["b"])
    g = _row2(p["bn"]["g"])
    bb = _row2(p["bn"]["b"])

    def h2_body(ea_ref, w1_ref, b1_ref, w2_ref, b2_ref, h2_ref):
        h1 = jnp.maximum(ea_ref[...] * w1_ref[...] + b1_ref[...], 0.0)
        h2_ref[...] = jnp.maximum(
            jnp.dot(h1, w2_ref[...], preferred_element_type=F32)
            + b2_ref[...], 0.0)

    wspecs = [
        pl.BlockSpec((1, H), lambda i: (0, 0)),
        pl.BlockSpec((1, H), lambda i: (0, 0)),
        pl.BlockSpec((H, H), lambda i: (0, 0)),
        pl.BlockSpec((1, H), lambda i: (0, 0)),
    ]
    h2 = pl.pallas_call(
        h2_body,
        grid=(ne // be,),
        in_specs=[pl.BlockSpec((be, 1), lambda i: (i, 0))] + wspecs,
        out_specs=pl.BlockSpec((be, H), lambda i: (i, 0)),
        out_shape=jax.ShapeDtypeStruct((ne, H), F32),
    )(ea, w1, b1, w2, b2)
    mu = jnp.mean(h2, axis=0, keepdims=True)
    var = jnp.var(h2, axis=0, keepdims=True)
    return _bn_norm(h2, mu, var, g, bb, split=True, br=be)


def _err_finalize(err_raw, rhs, h0):
    nc = rhs.shape[0]
    h1 = nc - h0

    def body(er_ref, rhs_ref, o_ref):
        row = jnp.concatenate([er_ref[0, 0:h0], er_ref[1, 0:h1]])
        o_ref[...] = row.reshape(nc, 1) - rhs_ref[...]

    return pl.pallas_call(
        body,
        out_shape=jax.ShapeDtypeStruct((nc, 1), F32),
    )(err_raw, rhs)


def _head(vs, params, br=1000):
    n = vs[0].shape[0]
    w1 = params["lin1"]["w"]
    b1 = _row2(params["lin1"]["b"])
    w2 = params["lin2"]["w"]
    b2 = _row2(params["lin2"]["b"])
    w3 = params["lin3"]["w"]
    b3 = _row2(params["lin3"]["b"])
    w4 = params["lin4"]["w"]
    b4 = _row2(params["lin4"]["b"])

    def body(v0, v1, v2, v3, v4, w1r, b1r, w2r, b2r, w3r, b3r, w4r, b4r,
             o_ref):
        xin = jnp.concatenate(
            [v0[...], v1[...], v2[...], v3[...], v4[...]], axis=1)
        x = jnp.dot(xin, w1r[...], preferred_element_type=F32) + b1r[...]
        x = jnp.maximum(x, 0.0)
        x = jnp.maximum(jnp.dot(x, w2r[...], preferred_element_type=F32)
                        + b2r[...], 0.0)
        x = jnp.maximum(jnp.dot(x, w3r[...], preferred_element_type=F32)
                        + b3r[...], 0.0)
        lg = jnp.dot(x, w4r[...], preferred_element_type=F32) + b4r[...]
        m = jnp.max(lg, axis=-1, keepdims=True)
        lse = m + jnp.log(jnp.sum(jnp.exp(lg - m), axis=-1, keepdims=True))
        o_ref[...] = lg - lse

    vspec = pl.BlockSpec((br, H), lambda i: (i, 0))
    return pl.pallas_call(
        body,
        grid=(n // br,),
        in_specs=[vspec] * 5 + [
            pl.BlockSpec((5 * H, H), lambda i: (0, 0)),
            pl.BlockSpec((1, H), lambda i: (0, 0)),
            pl.BlockSpec((H, H), lambda i: (0, 0)),
            pl.BlockSpec((1, H), lambda i: (0, 0)),
            pl.BlockSpec((H, H), lambda i: (0, 0)),
            pl.BlockSpec((1, H), lambda i: (0, 0)),
            pl.BlockSpec((H, 2), lambda i: (0, 0)),
            pl.BlockSpec((1, 2), lambda i: (0, 0)),
        ],
        out_specs=pl.BlockSpec((br, 2), lambda i: (i, 0)),
        out_shape=jax.ShapeDtypeStruct((n, 2), F32),
    )(*vs, w1, b1, w2, b2, w3, b3, w4, b4)


_CHUNK = 128
_NTILES = 16
_HC = H // 2


@functools.lru_cache(maxsize=None)
def _make_agg(n_src, n_out, ne, with_err):
    assert ne % _CHUNK == 0 and n_out % 2 == 0
    h0 = (n_out // 2 + 7) // 8 * 8
    h1 = n_out - h0
    acc_rows = ((h0 + _CHUNK) + _CHUNK - 1) // _CHUNK * _CHUNK
    n_chunks = ne // _CHUNK
    per_tile = -(-n_chunks // _NTILES)
    zch = acc_rows // _CHUNK
    full = min(h0, h1) // _CHUNK
    rem0 = h0 - full * _CHUNK
    rem1 = h1 - full * _CHUNK
    assert rem0 % 8 == 0 and rem1 % 8 == 0
    mesh = plsc.VectorSubcoreMesh(core_axis_name="c", subcore_axis_name="s")

    out_type = [jax.ShapeDtypeStruct((n_out, _HC), F32)]
    scratch = [
        pltpu.VMEM((1, _CHUNK), jnp.int32),
        pltpu.VMEM((1, _CHUNK), jnp.int32),
        pltpu.VMEM((_CHUNK, _HC), F32),
        pltpu.VMEM((_CHUNK, _HC), F32),
        pltpu.VMEM_SHARED((acc_rows, _HC), F32),
    ]
    if with_err:
        out_type.append(jax.ShapeDtypeStruct((2, zch, 1, _CHUNK), F32))
        scratch += [
            pltpu.VMEM((1, _CHUNK), F32),
            pltpu.VMEM((1, _CHUNK), F32),
            pltpu.VMEM_SHARED((acc_rows,), F32),
        ]

    def body(ns_hbm, ee_hbm, src_hbm, dst_hbm, *rest):
        if with_err:
            (aerr_hbm, ef_hbm, out_hbm, eout_hbm,
             isrc, idst, rows_v, msg_v, acc, ag_v, ef_v, eacc) = rest
        else:
            (out_hbm, isrc, idst, rows_v, msg_v, acc) = rest
        core = lax.axis_index("c")
        t = lax.axis_index("s")
        base_row = core * h0
        half_c = jnp.where(core == 0, h0, h1)

        def zrow(r, _):
            for v in range(_HC // 16):
                msg_v[r, pl.ds(v * 16, 16)] = jnp.zeros((16,), F32)
            return 0
        lax.fori_loop(0, _CHUNK, zrow, 0)

        def zacc(j, _):
            ch = t + _NTILES * j

            @pl.when(ch < zch)
            def _():
                pltpu.sync_copy(msg_v, acc.at[pl.ds(ch * _CHUNK, _CHUNK)])
                if with_err:
                    pltpu.sync_copy(msg_v.at[0],
                                    eacc.at[pl.ds(ch * _CHUNK, _HC)])
                    pltpu.sync_copy(msg_v.at[0],
                                    eacc.at[pl.ds(ch * _CHUNK + _HC, _HC)])
            return 0
        lax.fori_loop(0, -(-zch // _NTILES), zacc, 0)
        plsc.subcore_barrier()

        def step(j, _):
            ch = t + _NTILES * j

            @pl.when(ch < n_chunks)
            def _():
                eb = ch * _CHUNK
                pltpu.sync_copy(src_hbm.at[pl.ds(eb, _CHUNK)], isrc.at[0])
                pltpu.sync_copy(dst_hbm.at[pl.ds(eb, _CHUNK)], idst.at[0])
                pltpu.sync_copy(ee_hbm.at[pl.ds(eb, _CHUNK)], msg_v)
                for v in range(_CHUNK // 16):
                    d = idst[0, pl.ds(v * 16, 16)]
                    loc = d - base_row
                    oob = (loc < 0) | (loc >= half_c)
                    trash = half_c + v * 16 + lax.iota(jnp.int32, 16)
                    idst[0, pl.ds(v * 16, 16)] = jnp.where(oob, trash, loc)
                pltpu.sync_copy(ns_hbm.at[isrc.at[0]], rows_v)

                def crow(r, _):
                    for v in range(_HC // 16):
                        sl = pl.ds(v * 16, 16)
                        msg_v[r, sl] = jnp.maximum(
                            rows_v[r, sl] + msg_v[r, sl], 0.0)
                    return 0
                lax.fori_loop(0, _CHUNK, crow, 0)
                pltpu.sync_copy(msg_v, acc.at[idst.at[0]], add=True)

                if with_err:
                    pltpu.sync_copy(aerr_hbm.at[isrc.at[0]], ag_v.at[0])
                    pltpu.sync_copy(ef_hbm.at[pl.ds(eb, _CHUNK)], ef_v.at[0])
                    for v in range(_CHUNK // 16):
                        sl = pl.ds(v * 16, 16)
                        ag_v[0, sl] = ag_v[0, sl] * ef_v[0, sl]
                    pltpu.sync_copy(ag_v.at[0], eacc.at[idst.at[0]], add=True)
            return 0
        lax.fori_loop(0, per_tile, step, 0)
        plsc.subcore_barrier()

        def cout(j, _):
            ch = t + _NTILES * j

            @pl.when(ch < full)
            def _():
                pltpu.sync_copy(
                    acc.at[pl.ds(ch * _CHUNK, _CHUNK)],
                    out_hbm.at[pl.ds(base_row + ch * _CHUNK, _CHUNK)])
            if with_err:
                @pl.when(ch < zch)
                def _():
                    pltpu.sync_copy(eacc.at[pl.ds(ch * _CHUNK, _CHUNK)],
                                    eout_hbm.at[core, ch, 0])
            return 0
        lax.fori_loop(0, -(-zch // _NTILES), cout, 0)

        @pl.when(t == 0)
        def _():
            if rem0:
                @pl.when(core == 0)
                def _():
                    pltpu.sync_copy(
                        acc.at[pl.ds(full * _CHUNK, rem0)],
                        out_hbm.at[pl.ds(full * _CHUNK, rem0)])
            if rem1:
                @pl.when(core == 1)
                def _():
                    pltpu.sync_copy(
                        acc.at[pl.ds(full * _CHUNK, rem1)],
                        out_hbm.at[pl.ds(h0 + full * _CHUNK, rem1)])

    return pl.kernel(
        body, out_type=out_type, mesh=mesh, scratch_types=scratch,
        compiler_params=pltpu.CompilerParams(use_tc_tiling_on_sc=False))


def _agg(nsa, nsb, eea, eeb, src, dst, n_out, aerr=None, ef=None):
    n_src = nsa.shape[0]
    ne = src.shape[0]
    fb = _make_agg(n_src, n_out, ne, False)
    resb = fb(nsb, eeb, src, dst)
    tmpb = resb[0] if isinstance(resb, (list, tuple)) else resb
    if aerr is not None:
        fa = _make_agg(n_src, n_out, ne, True)
        tmpa, err_raw = fa(nsa, eea, src, dst, aerr, ef)
        return tmpa, tmpb, err_raw
    resa = fb(nsa, eea, src, dst)
    tmpa = resa[0] if isinstance(resa, (list, tuple)) else resa
    return tmpa, tmpb


def kernel(var_node_features, con_node_features, edge_index_var,
           edge_index_con, edge_features_var, edge_features_con, rhs,
           index, obj, params):
    nv = var_node_features.shape[0]
    nc = con_node_features.shape[0]

    src_v = edge_index_var[0]
    dst_v = edge_index_var[1]
    src_c = edge_index_con[0]
    dst_c = edge_index_con[1]
    efv1 = edge_features_var.reshape(-1)

    v0 = _mlp2(params["var_enc"], var_node_features)
    c0 = _mlp2(params["con_enc"], con_node_features)

    eev = [_edge_encode(params["vc%d" % k]["edge_encoder"],
                        edge_features_var) for k in range(1, 5)]
    eec = [_edge_encode(params["cv%d" % k]["edge_encoder"],
                        edge_features_con) for k in range(1, 5)]

    def seq_bn_joint(p, xa, xb, pad_out=False, ecol=None, relu_out=False):
        w1 = p["l1"]["w"]
        w2 = p["l2"]["w"]
        b2 = p["l2"]["b"]
        g = p["bn"]["g"]
        bb = p["bn"]["b"]
        if pad_out:
            z1 = jnp.zeros((H, 1), F32)
            z0 = jnp.zeros((1,), F32)
            w2 = jnp.concatenate([w2, z1], axis=1)
            b2 = jnp.concatenate([b2, z0])
            g = jnp.concatenate([g, z0])
            bb = jnp.concatenate([bb, z0])
        h2, _ = _seqbn_joint_stats(xa, xb, w1, _row2(p["l1"]["b"]),
                                   w2, _row2(b2))
        mu = jnp.mean(h2, axis=0, keepdims=True)
        var = jnp.var(h2, axis=0, keepdims=True)
        return _bn_norm(h2, mu, var, _row2(g), _row2(bb),
                        relu_out=relu_out, ecol=ecol, split=True)

    def seq_bn_mlp(p, eps, target, tmpa, tmpb):
        scal = (1.0 + eps).reshape(1, 1)
        h2, _ = _seqbn_mlp_stats(target, tmpa, tmpb, scal, p["l1"]["w"],
                                 _row2(p["l1"]["b"]), p["l2"]["w"],
                                 _row2(p["l2"]["b"]))
        mu = jnp.mean(h2, axis=0, keepdims=True)
        var = jnp.var(h2, axis=0, keepdims=True)
        return _bn_norm(h2, mu, var, _row2(p["bn"]["g"]),
                        _row2(p["bn"]["b"]), relu_out=True)

    v_prev, c_prev = v0, c0
    vs = [v0]
    for k in range(1, 5):
        pvc = params["vc%d" % k]
        pcv = params["cv%d" % k]
        pva = params["va%d" % k]

        a_k = _assign(pva, v_prev)
        if k == 2:
            a_err = _assign(params["va1"], v_prev)
        else:
            a_err = a_k

        nsa, nsb = seq_bn_joint(pvc["joint_var"], v_prev, a_k)
        tmp_ca, tmp_cb, err_raw = _agg(nsa, nsb, eev[k - 1][0],
                                       eev[k - 1][1], src_v, dst_v, nc,
                                       aerr=a_err.reshape(-1), ef=efv1)
        c_k = seq_bn_mlp(pvc["mlp"], pvc["eps"], c_prev, tmp_ca, tmp_cb)
        e_k = _err_finalize(err_raw.reshape(2, -1), rhs,
                            (nc // 2 + 7) // 8 * 8)

        jca, jcb = seq_bn_joint(pcv["joint_con"], c_k, e_k, pad_out=True,
                                ecol=e_k)
        tmp_va, tmp_vb = _agg(jca, jcb, eec[k - 1][0], eec[k - 1][1],
                              src_c, dst_c, nv)
        v_k = seq_bn_mlp(pcv["mlp"], pcv["eps"], v_prev, tmp_va, tmp_vb)

        vs.append(v_k)
        v_prev, c_prev = v_k, c_k

    return _head(vs, params)
